# baseline (speedup 1.0000x reference)
import functools

import jax
import jax.numpy as jnp
from jax import lax
from jax.experimental import pallas as pl
from jax.experimental.pallas import tpu as pltpu
from jax.experimental.pallas import tpu_sc as plsc

BSZ = 16384
NPOS = 52
HIDDEN = 128

_info = plsc.get_sparse_core_info()
_NC = _info.num_cores
_NS = _info.num_subcores
_NW = _NC * _NS
_ICH = 128
_BPW = BSZ // _NW
_NCHUNK = _BPW // _ICH

_PROW = 2
_PCOL = BSZ // _PROW

_sc_mesh = plsc.VectorSubcoreMesh(core_axis_name="c", subcore_axis_name="s")


@functools.partial(
    pl.kernel,
    mesh=_sc_mesh,
    out_type=jax.ShapeDtypeStruct((_PROW, _PCOL), jnp.int32),
    scratch_types=[
        pltpu.VMEM((_BPW,), jnp.int32),
        pltpu.VMEM((_BPW,), jnp.int32),
        pltpu.SemaphoreType.DMA,
    ],
)
def _pos_lookup(x_hbm, pos_hbm, out_hbm, xv, pv, sem):
    wid = lax.axis_index("s") * _NC + lax.axis_index("c")
    base = wid * _BPW
    pltpu.sync_copy(x_hbm.at[pl.ds(base, _BPW)], xv)
    copies = []
    for j in range(_NCHUNK):
        sl = pl.ds(j * _ICH, _ICH)
        copies.append(pltpu.async_copy(pos_hbm.at[xv.at[sl]], pv.at[sl], sem))
    for c in copies:
        c.wait()
    row = base // _PCOL
    col = base % _PCOL
    pltpu.sync_copy(pv, out_hbm.at[row, pl.ds(col, _BPW)])


def _expand_body(p_ref, c_ref, h_ref, oc_ref, oh_ref):
    i = pl.program_id(0)
    pl2 = p_ref[pl.ds(i, 1), :]
    onehot_t = (
        pl2 == lax.broadcasted_iota(jnp.int32, (NPOS, _PCOL), 0)
    ).astype(jnp.bfloat16)
    tab = jnp.concatenate([c_ref[...], h_ref[...]], axis=1)
    tab_hi = tab.astype(jnp.bfloat16)
    tab_lo = (tab - tab_hi.astype(jnp.float32)).astype(jnp.bfloat16)
    dn = (((0,), (0,)), ((), ()))
    d = lax.dot_general(
        onehot_t, tab_hi, dn, preferred_element_type=jnp.float32
    ) + lax.dot_general(
        onehot_t, tab_lo, dn, preferred_element_type=jnp.float32
    )
    oc_ref[...] = d[:, :HIDDEN]
    oh_ref[...] = d[:, HIDDEN:]


_expand = pl.pallas_call(
    _expand_body,
    grid=(_PROW,),
    in_specs=[
        pl.BlockSpec((_PROW, _PCOL), lambda i: (0, 0)),
        pl.BlockSpec((NPOS, HIDDEN), lambda i: (0, 0)),
        pl.BlockSpec((NPOS, HIDDEN), lambda i: (0, 0)),
    ],
    out_specs=[
        pl.BlockSpec((_PCOL, HIDDEN), lambda i: (i, 0)),
        pl.BlockSpec((_PCOL, HIDDEN), lambda i: (i, 0)),
    ],
    out_shape=[
        jax.ShapeDtypeStruct((BSZ, HIDDEN), jnp.float32),
        jax.ShapeDtypeStruct((BSZ, HIDDEN), jnp.float32),
    ],
)


def kernel(x, pos, c_table, h_table):
    p = _pos_lookup(x.astype(jnp.int32), pos.astype(jnp.int32))
    oc, oh = _expand(p, c_table, h_table)
    return oc.reshape(1, BSZ, HIDDEN), oh.reshape(1, BSZ, HIDDEN)


# speedup vs baseline: 1.0107x; 1.0107x over previous
import functools

import jax
import jax.numpy as jnp
from jax import lax
from jax.experimental import pallas as pl
from jax.experimental.pallas import tpu as pltpu
from jax.experimental.pallas import tpu_sc as plsc

BSZ = 16384
NPOS = 52
HIDDEN = 128

_info = plsc.get_sparse_core_info()
_NC = _info.num_cores
_NS = _info.num_subcores
_NW = _NC * _NS
_ICH = 128
_BPW = BSZ // _NW
_NCHUNK = _BPW // _ICH

_PROW = 4
_PCOL = BSZ // _PROW

_sc_mesh = plsc.VectorSubcoreMesh(core_axis_name="c", subcore_axis_name="s")


@functools.partial(
    pl.kernel,
    mesh=_sc_mesh,
    out_type=jax.ShapeDtypeStruct((_PROW, _PCOL), jnp.int32),
    scratch_types=[
        pltpu.VMEM((_BPW,), jnp.int32),
        pltpu.VMEM((_BPW,), jnp.int32),
        pltpu.SemaphoreType.DMA,
    ],
)
def _pos_lookup(x_hbm, pos_hbm, out_hbm, xv, pv, sem):
    wid = lax.axis_index("s") * _NC + lax.axis_index("c")
    base = wid * _BPW
    pltpu.sync_copy(x_hbm.at[pl.ds(base, _BPW)], xv)
    copies = []
    for j in range(_NCHUNK):
        sl = pl.ds(j * _ICH, _ICH)
        copies.append(pltpu.async_copy(pos_hbm.at[xv.at[sl]], pv.at[sl], sem))
    for c in copies:
        c.wait()
    row = base // _PCOL
    col = base % _PCOL
    pltpu.sync_copy(pv, out_hbm.at[row, pl.ds(col, _BPW)])


def _expand_body(p_ref, c_ref, h_ref, oc_ref, oh_ref):
    i = pl.program_id(0)
    pl2 = p_ref[pl.ds(i, 1), :]
    onehot_t = (
        pl2 == lax.broadcasted_iota(jnp.int32, (NPOS, _PCOL), 0)
    ).astype(jnp.bfloat16)
    tab = jnp.concatenate([c_ref[...], h_ref[...]], axis=1)
    tab_hi = tab.astype(jnp.bfloat16)
    tab_lo = (tab - tab_hi.astype(jnp.float32)).astype(jnp.bfloat16)
    dn = (((0,), (0,)), ((), ()))
    d = lax.dot_general(
        onehot_t, tab_hi, dn, preferred_element_type=jnp.float32
    ) + lax.dot_general(
        onehot_t, tab_lo, dn, preferred_element_type=jnp.float32
    )
    oc_ref[...] = d[:, :HIDDEN]
    oh_ref[...] = d[:, HIDDEN:]


_expand = pl.pallas_call(
    _expand_body,
    grid=(_PROW,),
    in_specs=[
        pl.BlockSpec((_PROW, _PCOL), lambda i: (0, 0)),
        pl.BlockSpec((NPOS, HIDDEN), lambda i: (0, 0)),
        pl.BlockSpec((NPOS, HIDDEN), lambda i: (0, 0)),
    ],
    out_specs=[
        pl.BlockSpec((_PCOL, HIDDEN), lambda i: (i, 0)),
        pl.BlockSpec((_PCOL, HIDDEN), lambda i: (i, 0)),
    ],
    out_shape=[
        jax.ShapeDtypeStruct((BSZ, HIDDEN), jnp.float32),
        jax.ShapeDtypeStruct((BSZ, HIDDEN), jnp.float32),
    ],
)


def kernel(x, pos, c_table, h_table):
    p = _pos_lookup(x.astype(jnp.int32), pos.astype(jnp.int32))
    oc, oh = _expand(p, c_table, h_table)
    return oc.reshape(1, BSZ, HIDDEN), oh.reshape(1, BSZ, HIDDEN)
